# baseline (speedup 1.0000x reference)
import functools

import jax
import jax.numpy as jnp
from jax import lax
from jax.experimental import pallas as pl
from jax.experimental.pallas import tpu as pltpu
from jax.experimental.pallas import tpu_sc as plsc

NUM_CORES = 2
NUM_SUBCORES = 16
NUM_WORKERS = NUM_CORES * NUM_SUBCORES
LANES = 16
CHUNK = 1280
NBUF = 2


def _embed_sc(xf, tab_flat, n_per_worker, dim):
    mesh = plsc.VectorSubcoreMesh(core_axis_name="c", subcore_axis_name="s")
    n = xf.shape[0]
    vd = tab_flat.shape[0]
    n_chunks = n_per_worker // CHUNK
    n_blocks = n_chunks // NBUF
    groups = CHUNK // LANES
    cd = CHUNK * dim

    @functools.partial(
        pl.kernel,
        out_type=jax.ShapeDtypeStruct((n * dim,), jnp.float32),
        mesh=mesh,
        scratch_types=[
            pltpu.VMEM((vd,), jnp.float32),
            pltpu.VMEM((CHUNK,), jnp.int32),
            pltpu.VMEM((CHUNK,), jnp.int32),
            pltpu.VMEM((cd,), jnp.float32),
            pltpu.VMEM((cd,), jnp.float32),
            pltpu.SemaphoreType.DMA,
            pltpu.SemaphoreType.DMA,
            pltpu.SemaphoreType.DMA,
            pltpu.SemaphoreType.DMA,
        ],
        compiler_params=pltpu.CompilerParams(needs_layout_passes=False, use_tc_tiling_on_sc=False),
    )
    def k(xf_hbm, tab_hbm, out_hbm, tab_v, idx0, idx1, out0, out1, si0, si1, so0, so1):
        idx_b = (idx0, idx1)
        out_b = (out0, out1)
        sem_i = (si0, si1)
        sem_o = (so0, so1)
        wid = lax.axis_index("s") * NUM_CORES + lax.axis_index("c")
        base = wid * n_per_worker
        pltpu.sync_copy(tab_hbm, tab_v)
        lane_row = lax.iota(jnp.int32, LANES) * dim

        for b in range(NBUF):
            pltpu.async_copy(
                xf_hbm.at[pl.ds(base + b * CHUNK, CHUNK)], idx_b[b], sem_i[b]
            )

        def blk_body(blk, carry):
            for b in range(NBUF):
                i = blk * NBUF + b
                off = base + i * CHUNK
                pltpu.make_async_copy(
                    xf_hbm.at[pl.ds(off, CHUNK)], idx_b[b], sem_i[b]
                ).wait()

                @pl.when(blk > 0)
                def _wait_out():
                    pltpu.make_async_copy(
                        out_b[b], out_hbm.at[pl.ds(0, cd)], sem_o[b]
                    ).wait()

                def grp(g, pos):
                    idxv = idx_b[b][pl.ds(g * LANES, LANES)]
                    rowbase = idxv * dim
                    vals = [
                        plsc.load_gather(tab_v, [rowbase + d]) for d in range(dim)
                    ]
                    for d in range(dim):
                        plsc.store_scatter(out_b[b], [pos + d], vals[d])
                    return pos + (LANES * dim)

                lax.fori_loop(0, groups, grp, lane_row)
                pltpu.async_copy(
                    out_b[b], out_hbm.at[pl.ds(off * dim, cd)], sem_o[b]
                )

                @pl.when(blk < n_blocks - 1)
                def _prefetch():
                    pltpu.async_copy(
                        xf_hbm.at[pl.ds(off + NBUF * CHUNK, CHUNK)],
                        idx_b[b],
                        sem_i[b],
                    )

            return carry

        lax.fori_loop(0, n_blocks, blk_body, 0)
        for b in range(NBUF):
            pltpu.make_async_copy(
                out_b[b], out_hbm.at[pl.ds(0, cd)], sem_o[b]
            ).wait()

    return k(xf, tab_flat)


def kernel(x, table):
    batch, seq = x.shape
    vocab, dim = table.shape
    n = batch * seq
    assert n % (NUM_WORKERS * CHUNK * NBUF) == 0
    n_per_worker = n // NUM_WORKERS
    xf = x.reshape(n).astype(jnp.int32)
    out = _embed_sc(xf, table.reshape(vocab * dim), n_per_worker, dim)
    return out.reshape(batch, seq, dim)


# speedup vs baseline: 1.7781x; 1.7781x over previous
import functools

import jax
import jax.numpy as jnp
from jax import lax
from jax.experimental import pallas as pl
from jax.experimental.pallas import tpu as pltpu
from jax.experimental.pallas import tpu_sc as plsc

NUM_CORES = 2
NUM_SUBCORES = 16
NUM_WORKERS = NUM_CORES * NUM_SUBCORES
LANES = 16
CHUNK = 1280
NBUF = 2


def _embed_sc(xf, tab_flat, n_per_worker, dim):
    mesh = plsc.VectorSubcoreMesh(core_axis_name="c", subcore_axis_name="s")
    n = xf.shape[0]
    vd = tab_flat.shape[0]
    n_chunks = n_per_worker // CHUNK
    n_blocks = n_chunks // NBUF
    cd = CHUNK * dim

    @functools.partial(
        pl.kernel,
        out_type=jax.ShapeDtypeStruct((n * dim,), jnp.float32),
        mesh=mesh,
        scratch_types=[
            pltpu.VMEM((vd,), jnp.float32),
            pltpu.VMEM((CHUNK,), jnp.int32),
            pltpu.VMEM((CHUNK,), jnp.int32),
            pltpu.VMEM((cd,), jnp.float32),
            pltpu.VMEM((cd,), jnp.float32),
            pltpu.SemaphoreType.DMA,
            pltpu.SemaphoreType.DMA,
            pltpu.SemaphoreType.DMA,
            pltpu.SemaphoreType.DMA,
        ],
        compiler_params=pltpu.CompilerParams(
            needs_layout_passes=False, use_tc_tiling_on_sc=False
        ),
    )
    def k(xf_hbm, tab_hbm, out_hbm, tab_v, idx0, idx1, out0, out1, si0, si1, so0, so1):
        idx_b = (idx0, idx1)
        out_b = (out0, out1)
        sem_i = (si0, si1)
        sem_o = (so0, so1)
        wid = lax.axis_index("s") * NUM_CORES + lax.axis_index("c")
        base = wid * n_per_worker
        pltpu.sync_copy(tab_hbm, tab_v)

        for b in range(NBUF):
            pltpu.async_copy(
                xf_hbm.at[pl.ds(base + b * CHUNK, CHUNK)], idx_b[b], sem_i[b]
            )

        def blk_body(blk, carry):
            for b in range(NBUF):
                i = blk * NBUF + b
                off = base + i * CHUNK
                pltpu.make_async_copy(
                    xf_hbm.at[pl.ds(off, CHUNK)], idx_b[b], sem_i[b]
                ).wait()

                @pl.when(blk > 0)
                def _wait_out():
                    pltpu.make_async_copy(
                        out_b[b], out_hbm.at[pl.ds(0, cd)], sem_o[b]
                    ).wait()

                def grp_body(g, c):
                    toffv = idx_b[b][pl.ds(g * LANES, LANES)] * dim
                    pos = g * (LANES * dim)
                    for j in range(LANES):
                        toff = toffv[j]
                        p = pos + j * dim
                        for h in range(dim // LANES):
                            out_b[b][pl.ds(p + h * LANES, LANES)] = tab_v[
                                pl.ds(toff + h * LANES, LANES)
                            ]
                    return c

                lax.fori_loop(0, CHUNK // LANES, grp_body, 0)
                pltpu.async_copy(
                    out_b[b], out_hbm.at[pl.ds(off * dim, cd)], sem_o[b]
                )

                @pl.when(blk < n_blocks - 1)
                def _prefetch():
                    pltpu.async_copy(
                        xf_hbm.at[pl.ds(off + NBUF * CHUNK, CHUNK)],
                        idx_b[b],
                        sem_i[b],
                    )

            return carry

        lax.fori_loop(0, n_blocks, blk_body, 0)
        for b in range(NBUF):
            pltpu.make_async_copy(
                out_b[b], out_hbm.at[pl.ds(0, cd)], sem_o[b]
            ).wait()

    return k(xf, tab_flat)


def kernel(x, table):
    batch, seq = x.shape
    vocab, dim = table.shape
    n = batch * seq
    assert n % (NUM_WORKERS * CHUNK * NBUF) == 0
    n_per_worker = n // NUM_WORKERS
    xf = x.reshape(n).astype(jnp.int32)
    out = _embed_sc(xf, table.reshape(vocab * dim), n_per_worker, dim)
    return out.reshape(batch, seq, dim)


# speedup vs baseline: 3.1197x; 1.7546x over previous
import functools

import jax
import jax.numpy as jnp
from jax import lax
from jax.experimental import pallas as pl
from jax.experimental.pallas import tpu as pltpu
from jax.experimental.pallas import tpu_sc as plsc

NUM_CORES = 2
NUM_SUBCORES = 16
NUM_WORKERS = NUM_CORES * NUM_SUBCORES
LANES = 16
CHUNK = 320
NBUF = 2


def _embed_sc(xf, tab_flat, n_per_worker, dim):
    mesh = plsc.VectorSubcoreMesh(core_axis_name="c", subcore_axis_name="s")
    n = xf.shape[0]
    vd = tab_flat.shape[0]
    n_chunks = n_per_worker // CHUNK
    n_blocks = n_chunks // NBUF
    cd = CHUNK * dim

    @functools.partial(
        pl.kernel,
        out_type=jax.ShapeDtypeStruct((n, dim), jnp.float32),
        mesh=mesh,
        scratch_types=[
            pltpu.VMEM((vd,), jnp.float32),
            pltpu.VMEM((CHUNK,), jnp.int32),
            pltpu.VMEM((CHUNK,), jnp.int32),
            pltpu.VMEM((CHUNK, dim), jnp.float32),
            pltpu.VMEM((CHUNK, dim), jnp.float32),
            pltpu.SemaphoreType.DMA,
            pltpu.SemaphoreType.DMA,
            pltpu.SemaphoreType.DMA,
            pltpu.SemaphoreType.DMA,
        ],
        compiler_params=pltpu.CompilerParams(needs_layout_passes=False),
    )
    def k(xf_hbm, tab_hbm, out_hbm, tab_v, idx0, idx1, out0, out1, si0, si1, so0, so1):
        idx_b = (idx0, idx1)
        out_b = (out0, out1)
        sem_i = (si0, si1)
        sem_o = (so0, so1)
        wid = lax.axis_index("s") * NUM_CORES + lax.axis_index("c")
        base = wid * n_per_worker
        pltpu.sync_copy(tab_hbm, tab_v)

        for b in range(NBUF):
            pltpu.async_copy(
                xf_hbm.at[pl.ds(base + b * CHUNK, CHUNK)], idx_b[b], sem_i[b]
            )

        def blk_body(blk, carry):
            for b in range(NBUF):
                i = blk * NBUF + b
                off = base + i * CHUNK
                pltpu.make_async_copy(
                    xf_hbm.at[pl.ds(off, CHUNK)], idx_b[b], sem_i[b]
                ).wait()

                @pl.when(blk > 0)
                def _wait_out():
                    pltpu.make_async_copy(
                        out_b[b], out_hbm.at[pl.ds(0, CHUNK), :], sem_o[b]
                    ).wait()

                def grp_body(g, c):
                    toffv = idx_b[b][pl.ds(g * LANES, LANES)] * dim
                    for j in range(LANES):
                        toff = toffv[j]
                        rr = g * LANES + j
                        for h in range(dim // LANES):
                            out_b[b][rr, pl.ds(h * LANES, LANES)] = tab_v[
                                pl.ds(toff + h * LANES, LANES)
                            ]
                    return c

                lax.fori_loop(0, CHUNK // LANES, grp_body, 0)
                pltpu.async_copy(
                    out_b[b], out_hbm.at[pl.ds(off, CHUNK), :], sem_o[b]
                )

                @pl.when(blk < n_blocks - 1)
                def _prefetch():
                    pltpu.async_copy(
                        xf_hbm.at[pl.ds(off + NBUF * CHUNK, CHUNK)],
                        idx_b[b],
                        sem_i[b],
                    )

            return carry

        lax.fori_loop(0, n_blocks, blk_body, 0)
        for b in range(NBUF):
            pltpu.make_async_copy(
                out_b[b], out_hbm.at[pl.ds(0, CHUNK), :], sem_o[b]
            ).wait()

    return k(xf, tab_flat)


def kernel(x, table):
    batch, seq = x.shape
    vocab, dim = table.shape
    n = batch * seq
    assert n % (NUM_WORKERS * CHUNK * NBUF) == 0
    n_per_worker = n // NUM_WORKERS
    xf = x.reshape(n).astype(jnp.int32)
    out = _embed_sc(xf, table.reshape(vocab * dim), n_per_worker, dim)
    return out.reshape(batch, seq, dim)
